# Initial kernel scaffold; baseline (speedup 1.0000x reference)
import jax
import jax.numpy as jnp
from jax.experimental import pallas as pl


def kernel(edges_flat, cu_seqlens, generated_data, target_quantile):
    raise NotImplementedError("write your pallas kernel here")


# speedup vs baseline: 27.7093x; 27.7093x over previous
import functools

import jax
import jax.numpy as jnp
from jax import lax
from jax.experimental import pallas as pl
from jax.experimental.pallas import tpu as pltpu
from jax.experimental.pallas import tpu_sc as plsc

NC, NS, L = 2, 16, 16
NW = NC * NS
MAXL = 200
NN = 16384
TE = 1638400
WPN = 3 * MAXL
NPW = NN // NW
SPS = NN // NS
NV = SPS // L

_mesh = plsc.VectorSubcoreMesh(
    core_axis_name="c", subcore_axis_name="s", num_cores=NC, num_subcores=NS
)
_params = pltpu.CompilerParams(needs_layout_passes=False)

_I31 = 0x7FFFFFFF


def _lane():
    return lax.iota(jnp.int32, L)


@functools.partial(
    pl.kernel,
    out_type=jax.ShapeDtypeStruct((2 * NN,), jnp.int32),
    mesh=_mesh,
    compiler_params=_params,
    scratch_types=[
        pltpu.VMEM((SPS,), jnp.float32),
        pltpu.VMEM((SPS,), jnp.int32),
        pltpu.VMEM((SPS,), jnp.int32),
        pltpu.VMEM((SPS,), jnp.int32),
        pltpu.VMEM((SPS,), jnp.int32),
        pltpu.VMEM((SPS,), jnp.int32),
        pltpu.VMEM((SPS,), jnp.int32),
        pltpu.VMEM_SHARED((NN,), jnp.int32),
        pltpu.VMEM_SHARED((NN,), jnp.int32),
    ],
)
def _sort_kernel(comb_hbm, out_hbm,
                 fb, ka, va, kb, vb, pk, pv, shk, shv):
    c = lax.axis_index("c")
    s = lax.axis_index("s")
    base = s * SPS
    cbase = c * NN + base
    lane = _lane()

    pltpu.sync_copy(comb_hbm.at[pl.ds(cbase, SPS)], fb)

    for t in range(NV):
        i = plsc.bitcast(fb[pl.ds(t * L, L)], jnp.int32)
        ka[pl.ds(t * L, L)] = jnp.where(i >= 0, i, i ^ _I31)
        va[pl.ds(t * L, L)] = lane + (base + t * L)

    def do_pass(kk, jj, src_k, src_v, dst_k, dst_v):
        @pl.when(jj >= SPS)
        def _():
            pltpu.sync_copy(src_k, shk.at[pl.ds(base, SPS)])
            pltpu.sync_copy(src_v, shv.at[pl.ds(base, SPS)])
            plsc.subcore_barrier()
            ps = s ^ (jj // SPS)
            pltpu.sync_copy(shk.at[pl.ds(ps * SPS, SPS)], pk)
            pltpu.sync_copy(shv.at[pl.ds(ps * SPS, SPS)], pv)
            is_lo = (s & (jj // SPS)) == 0
            asc = (base & kk) == 0
            take_min = is_lo == asc

            def t_body(t, _):
                sl = pl.ds(t * L, L)
                xk, xv = src_k[sl], src_v[sl]
                yk, yv = pk[sl], pv[sl]
                ltv = (xk < yk) | ((xk == yk) & (xv < yv))
                cond = ltv == take_min
                dst_k[sl] = jnp.where(cond, xk, yk)
                dst_v[sl] = jnp.where(cond, xv, yv)
                return 0

            lax.fori_loop(0, NV, t_body, 0)
            plsc.subcore_barrier()

        @pl.when((jj >= L) & (jj < SPS))
        def _():
            jv = jj // L

            def q_body(q, _):
                lo = q & (jv - 1)
                t = (q - lo) * 2 + lo
                t2 = t + jv
                s1 = pl.ds(t * L, L)
                s2 = pl.ds(t2 * L, L)
                asc = ((base + t * L) & kk) == 0
                xk, xv = src_k[s1], src_v[s1]
                yk, yv = src_k[s2], src_v[s2]
                ltv = (xk < yk) | ((xk == yk) & (xv < yv))
                cond = ltv == asc
                dst_k[s1] = jnp.where(cond, xk, yk)
                dst_v[s1] = jnp.where(cond, xv, yv)
                dst_k[s2] = jnp.where(cond, yk, xk)
                dst_v[s2] = jnp.where(cond, yv, xv)
                return 0

            lax.fori_loop(0, NV // 2, q_body, 0)

        @pl.when(jj < L)
        def _():
            perm = lane ^ jj
            is_lo = (lane & jj) == 0

            def t_body(t, _):
                sl = pl.ds(t * L, L)
                xk, xv = src_k[sl], src_v[sl]
                gi = perm + t * L
                yk = plsc.load_gather(src_k, [gi])
                yv = plsc.load_gather(src_v, [gi])
                asc = (((base + t * L) + lane) & kk) == 0
                take_min = is_lo == asc
                ltv = (xk < yk) | ((xk == yk) & (xv < yv))
                cond = ltv == take_min
                dst_k[sl] = jnp.where(cond, xk, yk)
                dst_v[sl] = jnp.where(cond, xv, yv)
                return 0

            lax.fori_loop(0, NV, t_body, 0)

    def stage_body(st_i, ph):
        kk = lax.shift_left(jnp.int32(1), st_i)

        def j_body(m, ph):
            jj = lax.shift_left(jnp.int32(1), st_i - 1 - m)

            @pl.when(ph == 0)
            def _():
                do_pass(kk, jj, ka, va, kb, vb)

            @pl.when(ph == 1)
            def _():
                do_pass(kk, jj, kb, vb, ka, va)

            return ph ^ 1

        return lax.fori_loop(0, st_i, j_body, ph)

    lax.fori_loop(1, 15, stage_body, jnp.int32(0))

    for t in range(NV):
        sl = pl.ds(t * L, L)
        k = kb[sl]
        bits = jnp.where(k >= 0, k, k ^ _I31)
        pk[sl] = jnp.where(c == 0, vb[sl], bits)
    pltpu.sync_copy(pk, out_hbm.at[pl.ds(cbase, SPS)])


@functools.partial(
    pl.kernel,
    out_type=jax.ShapeDtypeStruct((NN,), jnp.int32),
    mesh=_mesh,
    compiler_params=_params,
    scratch_types=[
        pltpu.VMEM((NN,), jnp.float32),
        pltpu.VMEM((NPW // 128, 128), jnp.int32),
        pltpu.VMEM((NPW,), jnp.int32),
        pltpu.SemaphoreType.DMA,
    ],
)
def _final_kernel(gidx_hbm, st_hbm, map_hbm, st_v, gi_v, val_v, sem):
    c = lax.axis_index("c")
    s = lax.axis_index("s")
    wid = s * NC + c
    base = wid * NPW
    lane = _lane()

    pltpu.sync_copy(st_hbm, st_v)
    pltpu.sync_copy(gidx_hbm.at[pl.ds(wid * (NPW // 128), NPW // 128)], gi_v)

    nm1 = jnp.float32(NN - 1)
    for t in range(NPW // L):
        r = (base + t * L) + lane
        ii = (r.astype(jnp.float32) / nm1) * nm1
        fl = ii.astype(jnp.int32)
        flf = fl.astype(jnp.float32)
        ce = jnp.minimum(fl + (ii > flf).astype(jnp.int32), NN - 1)
        wc = ii - flf
        a = plsc.load_gather(st_v, [fl])
        b = plsc.load_gather(st_v, [ce])
        val = (jnp.float32(1.0) - wc) * a + wc * b
        val_v[pl.ds(t * L, L)] = val.astype(jnp.int32)

    for q in range(NPW // 128):
        pltpu.async_copy(
            val_v.at[pl.ds(q * 128, 128)], map_hbm.at[gi_v.at[q]], sem
        ).wait()


@functools.partial(
    pl.kernel,
    out_type=(
        jax.ShapeDtypeStruct((NN * WPN,), jnp.float32),
        jax.ShapeDtypeStruct((NN * MAXL,), jnp.int32),
        jax.ShapeDtypeStruct((NN,), jnp.int32),
    ),
    mesh=_mesh,
    compiler_params=_params,
    scratch_types=[
        pltpu.VMEM((NPW + 16,), jnp.int32),
        pltpu.VMEM((1216,), jnp.float32),
        pltpu.VMEM((640,), jnp.float32),
        pltpu.VMEM((208,), jnp.int32),
        pltpu.VMEM((NPW,), jnp.int32),
    ],
)
def _pad_kernel(ef_hbm, cu_hbm, pad_hbm, int_hbm, len_hbm,
                cu_v, in_v, out_v, int_v, len_v):
    c = lax.axis_index("c")
    s = lax.axis_index("s")
    wid = s * NC + c
    base = wid * NPW
    lane = _lane()

    pltpu.sync_copy(cu_hbm.at[pl.ds(base, NPW + 16)], cu_v)

    def body(n, _):
        cuv = cu_v[pl.ds(n, L)]
        start = cuv[0]
        end = cuv[1]
        ln = end - start
        len_c = jnp.minimum(ln, MAXL)
        add1 = jnp.where(ln <= MAXL, jnp.float32(1.0), jnp.float32(0.0))
        w0 = start * 3
        a0 = jnp.minimum(w0 - (w0 & 7), 3 * TE - 608)
        a0 = pl.multiple_of(a0, 8)
        d = w0 - a0
        pltpu.sync_copy(ef_hbm.at[pl.ds(a0, 608)], in_v.at[pl.ds(0, 608)])
        for t in range(38):
            p = (lane + t * L) // 3
            x = in_v[pl.ds(d + t * L, L)]
            out_v[pl.ds(t * L, L)] = jnp.where(p < len_c, x + add1,
                                               jnp.float32(0.0))
        for t in range(13):
            gi = lane * 3 + (2 + t * 3 * L)
            v = plsc.load_gather(out_v, [gi])
            int_v[pl.ds(t * L, L)] = v.astype(jnp.int32)
        g = base + n
        pltpu.sync_copy(out_v.at[pl.ds(0, WPN)],
                        pad_hbm.at[pl.ds(g * WPN, WPN)])
        pltpu.sync_copy(int_v.at[pl.ds(0, MAXL)],
                        int_hbm.at[pl.ds(g * MAXL, MAXL)])
        return 0

    lax.fori_loop(0, NPW, body, 0)
    for t in range(NPW // L):
        starts = cu_v[pl.ds(t * L, L)]
        ends = cu_v[pl.ds(t * L + 1, L)]
        len_v[pl.ds(t * L, L)] = jnp.minimum(ends - starts, MAXL)
    pltpu.sync_copy(len_v, len_hbm.at[pl.ds(base, NPW)])


@jax.jit
def kernel(edges_flat, cu_seqlens, generated_data, target_quantile):
    ef = edges_flat.reshape(-1)
    cu_pad = jnp.concatenate(
        [cu_seqlens, jnp.full((31,), TE, jnp.int32)])
    gen = generated_data.reshape(-1)

    comb = jnp.concatenate([gen, target_quantile])
    sorted_comb = _sort_kernel(comb)
    gidx = sorted_comb[:NN]
    st = lax.bitcast_convert_type(sorted_comb[NN:], jnp.float32)
    mapped = _final_kernel(gidx.reshape(NN // 128, 128), st)
    padded_flat, int_flat, lens = _pad_kernel(ef, cu_pad)

    return (padded_flat.reshape(NN, MAXL, 3), lens,
            int_flat.reshape(NN, MAXL), mapped.reshape(NN, 1))
